# baseline (speedup 1.0000x reference)
import functools

import jax
import jax.numpy as jnp
from jax import lax
from jax.experimental import pallas as pl
from jax.experimental.pallas import tpu as pltpu
from jax.experimental.pallas import tpu_sc as plsc

MAXLEN = 8192
DM = 768
BATCH = 4
HALF = MAXLEN // 2
NC = 2
NS = 16
NW = NC * NS
K = HALF // NW
C = 64
L = 16

_mesh = plsc.VectorSubcoreMesh(core_axis_name="c", subcore_axis_name="s")


@functools.partial(
    pl.kernel,
    mesh=_mesh,
    out_type=jax.ShapeDtypeStruct((BATCH * MAXLEN, DM), jnp.float32),
    scratch_types=[
        pltpu.VMEM((C, DM), jnp.float32),
        pltpu.VMEM((C, DM), jnp.float32),
        pltpu.VMEM((C,), jnp.int32),
        pltpu.SemaphoreType.DMA,
    ],
)
def _emb(table_hbm, out_hbm, fwd_v, rev_v, idx_v, sem):
    wid = lax.axis_index("s") * NC + lax.axis_index("c")
    lane = lax.iota(jnp.int32, L)
    for c in range(K // C):
        s = wid * K + c * C
        pltpu.sync_copy(table_hbm.at[pl.ds(s, C)], fwd_v)
        for j in range(C // L):
            idx_v[pl.ds(j * L, L)] = (HALF - s - j * L) - lane
        pltpu.async_copy(table_hbm.at[idx_v], rev_v, sem).wait()

        copies = []
        for b in range(BATCH):
            copies.append(
                pltpu.async_copy(
                    fwd_v, out_hbm.at[pl.ds(b * MAXLEN + HALF + s, C)], sem
                )
            )
            copies.append(
                pltpu.async_copy(
                    rev_v, out_hbm.at[pl.ds(b * MAXLEN + s, C)], sem
                )
            )
        for cp in copies:
            cp.wait()


def kernel(x, table):
    del x
    return _emb(table).reshape(BATCH, MAXLEN, DM)


# speedup vs baseline: 1.0951x; 1.0951x over previous
import functools

import jax
import jax.numpy as jnp
from jax import lax
from jax.experimental import pallas as pl
from jax.experimental.pallas import tpu as pltpu
from jax.experimental.pallas import tpu_sc as plsc

MAXLEN = 8192
DM = 768
BATCH = 4
HALF = MAXLEN // 2
NC = 2
NS = 16
NW = NC * NS
K = HALF // NW
C = 64
L = 16

_mesh = plsc.VectorSubcoreMesh(core_axis_name="c", subcore_axis_name="s")


@functools.partial(
    pl.kernel,
    mesh=_mesh,
    out_type=jax.ShapeDtypeStruct((BATCH * MAXLEN, DM), jnp.float32),
    scratch_types=[
        pltpu.VMEM((2, C, DM), jnp.float32),
        pltpu.VMEM((2 * BATCH, C), jnp.int32),
        pltpu.VMEM((1, DM), jnp.float32),
        pltpu.SemaphoreType.DMA,
        pltpu.SemaphoreType.DMA,
    ],
)
def _emb(table_hbm, out_hbm, rows_v, idx_v, row0_v, ssem, wsem):
    wid = lax.axis_index("s") * NC + lax.axis_index("c")
    lane = lax.iota(jnp.int32, L)

    for p in range(2):
        for b in range(BATCH):
            base = (b * MAXLEN + HALF) - (wid * K + p * C)
            for j in range(C // L):
                idx_v[p * BATCH + b, pl.ds(j * L, L)] = (base - j * L) - lane

    st0 = pltpu.async_copy(table_hbm.at[pl.ds(wid * K, C)], rows_v.at[0], ssem)
    st0.wait()
    st1 = pltpu.async_copy(table_hbm.at[pl.ds(wid * K + C, C)], rows_v.at[1], ssem)

    writes = []
    for p in range(2):
        if p == 1:
            st1.wait()
        s = wid * K + p * C
        for b in range(BATCH):
            writes.append(
                pltpu.async_copy(
                    rows_v.at[p], out_hbm.at[pl.ds(b * MAXLEN + HALF + s, C)], wsem
                )
            )
            writes.append(
                pltpu.async_copy(rows_v.at[p], out_hbm.at[idx_v.at[p * BATCH + b]], wsem)
            )

    @pl.when(wid == 0)
    def _():
        pltpu.sync_copy(table_hbm.at[pl.ds(HALF, 1)], row0_v)
        for b in range(BATCH):
            pltpu.sync_copy(row0_v, out_hbm.at[pl.ds(b * MAXLEN, 1)])

    for cp in writes:
        cp.wait()


def kernel(x, table):
    del x
    return _emb(table).reshape(BATCH, MAXLEN, DM)


# speedup vs baseline: 1.1114x; 1.0149x over previous
import functools

import jax
import jax.numpy as jnp
from jax import lax
from jax.experimental import pallas as pl
from jax.experimental.pallas import tpu as pltpu
from jax.experimental.pallas import tpu_sc as plsc

MAXLEN = 8192
DM = 768
BATCH = 4
HALF = MAXLEN // 2
NC = 2
NS = 16
NW = NC * NS
K = HALF // NW
C = 64
L = 16

_mesh = plsc.VectorSubcoreMesh(core_axis_name="c", subcore_axis_name="s")


@functools.partial(
    pl.kernel,
    mesh=_mesh,
    out_type=jax.ShapeDtypeStruct((BATCH * MAXLEN, DM), jnp.float32),
    scratch_types=[
        pltpu.VMEM((2, C, DM), jnp.float32),
        pltpu.VMEM((2 * BATCH, C), jnp.int32),
        pltpu.VMEM((1, DM), jnp.float32),
        pltpu.SemaphoreType.DMA,
        pltpu.SemaphoreType.DMA,
    ],
)
def _emb(table_hbm, out_hbm, rows_v, idx_v, row0_v, ssem, wsem):
    wid = lax.axis_index("s") * NC + lax.axis_index("c")
    lane = lax.iota(jnp.int32, L)

    st0 = pltpu.async_copy(table_hbm.at[pl.ds(wid * K, C)], rows_v.at[0], ssem)
    st1 = pltpu.async_copy(table_hbm.at[pl.ds(wid * K + C, C)], rows_v.at[1], ssem)

    for p in range(2):
        for b in range(BATCH):
            base = (b * MAXLEN + HALF) - (wid * K + p * C)
            for j in range(C // L):
                idx_v[p * BATCH + b, pl.ds(j * L, L)] = (base - j * L) - lane

    writes = []
    for p in range(2):
        (st0 if p == 0 else st1).wait()
        s = wid * K + p * C
        for b in range(BATCH):
            writes.append(
                pltpu.async_copy(
                    rows_v.at[p], out_hbm.at[pl.ds(b * MAXLEN + HALF + s, C)], wsem
                )
            )
            writes.append(
                pltpu.async_copy(rows_v.at[p], out_hbm.at[idx_v.at[p * BATCH + b]], wsem)
            )

    @pl.when(wid == 0)
    def _():
        pltpu.sync_copy(table_hbm.at[pl.ds(HALF, 1)], row0_v)
        for b in range(BATCH):
            pltpu.sync_copy(row0_v, out_hbm.at[pl.ds(b * MAXLEN, 1)])

    for cp in writes:
        cp.wait()


def kernel(x, table):
    del x
    return _emb(table).reshape(BATCH, MAXLEN, DM)
